# baseline (speedup 1.0000x reference)
import functools

import jax
import jax.numpy as jnp
from jax import lax
from jax.experimental import pallas as pl
from jax.experimental.pallas import tpu as pltpu
from jax.experimental.pallas import tpu_sc as plsc

_HIDDEN = 64
_SCALE = 8.0


@functools.cache
def _make_lookup(B, V2):
    info = plsc.get_sparse_core_info()
    NC, NS, L = info.num_cores, info.num_subcores, info.num_lanes
    NW = NC * NS
    assert B % NW == 0
    b_per_w = B // NW
    C = 400
    n_chunks = b_per_w // C
    assert b_per_w % C == 0

    mesh = plsc.VectorSubcoreMesh(core_axis_name="c", subcore_axis_name="s")

    @functools.partial(
        pl.kernel,
        out_type=jax.ShapeDtypeStruct((B, _HIDDEN), jnp.float32),
        mesh=mesh,
        scratch_types=[
            pltpu.VMEM((C,), jnp.int32),
            pltpu.VMEM((C,), jnp.int32),
            pltpu.VMEM((C, 2 * _HIDDEN), jnp.float32),
            pltpu.VMEM((C, _HIDDEN), jnp.float32),
            pltpu.SemaphoreType.DMA,
        ],
    )
    def lookup(idx_hbm, t2_hbm, out_hbm, idx_v, pidx_v, pairs_v,
               stage_v, sem):
        wid = lax.axis_index("s") * NC + lax.axis_index("c")
        base = wid * b_per_w

        def chunk_body(c, carry):
            start = base + c * C
            pltpu.sync_copy(idx_hbm.at[pl.ds(start, C)], idx_v)

            def halve(g, carry2):
                v = idx_v[pl.ds(g * L, L)]
                pidx_v[pl.ds(g * L, L)] = v >> 1
                return carry2

            lax.fori_loop(0, C // L, halve, 0)
            pltpu.async_copy(t2_hbm.at[pidx_v], pairs_v, sem).wait()

            def select_group(g, carry2):
                idx16 = idx_v[pl.ds(g * L, L)]
                off16 = (idx16 & 1) * _HIDDEN
                for l in range(L):
                    off = off16[l]
                    i = g * L + l
                    for j in range(_HIDDEN // L):
                        stage_v[i, pl.ds(j * L, L)] = (
                            pairs_v[i, pl.ds(off + j * L, L)] * _SCALE
                        )
                return carry2

            lax.fori_loop(0, C // L, select_group, 0)
            pltpu.sync_copy(stage_v, out_hbm.at[pl.ds(start, C)])
            return carry

        lax.fori_loop(0, n_chunks, chunk_body, 0)

    return lookup


def kernel(x, table):
    B = x.shape[0] * x.shape[1]
    flat = x.reshape(B).astype(jnp.int32)
    t2 = table.reshape(table.shape[0] // 2, 2 * _HIDDEN)
    out = _make_lookup(B, t2.shape[0])(flat, t2)
    return out.reshape(x.shape[0], x.shape[1], _HIDDEN)
